# Initial kernel scaffold; baseline (speedup 1.0000x reference)
import jax
import jax.numpy as jnp
from jax.experimental import pallas as pl


def kernel(states, actions, agent_qs, max_filter, target, W1, b1, W2, b2, W3, b3):
    raise NotImplementedError("write your pallas kernel here")


# speedup vs baseline: 3.7423x; 3.7423x over previous
import functools

import jax
import jax.numpy as jnp
import numpy as np
from jax.experimental import pallas as pl
from jax.experimental.pallas import tpu as pltpu

_N_AGENTS = 8
_N_ACTIONS = 14
_STATE_DIM = 200
_SAMPLE = 16
_EMBED = 512
_BB = 128
_ROWS = _BB * _N_AGENTS


def _fwd_kernel(states_ref, gcjk_ref, inv_ref, acts2_ref, ind_ref, qs_ref,
                mf_ref, w1s_ref, w1n_ref, w1i_ref, selrep_ref, b1_ref,
                b2_ref, w2_ref, w3t_ref, b3_ref, west_ref, qtot_ref):
    i = pl.program_id(0)
    bb, na, sdim = _BB, _N_AGENTS, _STATE_DIM

    mx = jnp.max(states_ref[...])

    gcjk = gcjk_ref[...]
    invb = inv_ref[...]
    invjk = jnp.broadcast_to(invb[:, None, :], (bb, na, 128)).reshape(_ROWS, 128)
    prod = gcjk * (invjk < gcjk).astype(jnp.float32)
    mrep = jnp.dot(prod, selrep_ref[...],
                   preferred_element_type=jnp.float32)

    acts2 = acts2_ref[...]
    actsb = jnp.broadcast_to(acts2[:, None, :], (bb, na, 112)).reshape(_ROWS, 112)
    hn_in = mrep * actsb

    st = states_ref[pl.ds(i * bb, bb), :] / mx
    hs = jnp.dot(st, w1s_ref[...], preferred_element_type=jnp.float32)
    hsb = jnp.broadcast_to(hs[:, None, :], (bb, na, _EMBED)).reshape(_ROWS, _EMBED)
    hn = jnp.dot(hn_in, w1n_ref[...], preferred_element_type=jnp.float32)
    hi = jnp.dot(ind_ref[...], w1i_ref[...], preferred_element_type=jnp.float32)
    h1 = jnp.maximum(hsb + hn + hi + b1_ref[...], 0.0)

    h2 = jnp.maximum(
        jnp.dot(h1, w2_ref[...], preferred_element_type=jnp.float32) + b2_ref[...],
        0.0)

    h23 = h2.reshape(bb, na, _EMBED)
    w = jnp.sum(h23 * w3t_ref[...][None, :, :], axis=2) + b3_ref[0, 0]
    west_ref[...] = w

    qs = qs_ref[...]
    mf = mf_ref[...]
    qtot_ref[...] = jnp.mean((w * (1.0 - mf) + mf) * qs, axis=1,
                             keepdims=True)


@functools.partial(jax.jit, static_argnames=())
def _run(states2, gcjk, inv128, acts2, ind16, qs2, mf2, w1s, w1n_exp, w1i16,
         selrep, b1r, b2r, w2, w3t, b3r):
    bs = states2.shape[0]
    nb = bs // _BB
    rows = bs * _N_AGENTS

    full = lambda shape: pl.BlockSpec(shape, lambda i: (0,) * len(shape))
    west, qtot = pl.pallas_call(
        _fwd_kernel,
        grid=(nb,),
        in_specs=[
            full((bs, _STATE_DIM)),
            pl.BlockSpec((_ROWS, 128), lambda i: (i, 0)),
            pl.BlockSpec((_BB, 128), lambda i: (i, 0)),
            pl.BlockSpec((_BB, 112), lambda i: (i, 0)),
            pl.BlockSpec((_ROWS, 16), lambda i: (i, 0)),
            pl.BlockSpec((_BB, _N_AGENTS), lambda i: (i, 0)),
            pl.BlockSpec((_BB, _N_AGENTS), lambda i: (i, 0)),
            full((_STATE_DIM, _EMBED)),
            full((112, _EMBED)),
            full((16, _EMBED)),
            full((128, 112)),
            full((1, _EMBED)),
            full((1, _EMBED)),
            full((_EMBED, _EMBED)),
            full((1, _EMBED)),
            full((1, 1)),
        ],
        out_specs=[
            pl.BlockSpec((_BB, _N_AGENTS), lambda i: (i, 0)),
            pl.BlockSpec((_BB, 1), lambda i: (i, 0)),
        ],
        out_shape=[
            jax.ShapeDtypeStruct((bs, _N_AGENTS), jnp.float32),
            jax.ShapeDtypeStruct((bs, 1), jnp.float32),
        ],
    )(states2, gcjk, inv128, acts2, ind16, qs2, mf2, w1s, w1n_exp, w1i16,
      selrep, b1r, b2r, w2, w3t, b3r)
    return west, qtot


def kernel(states, actions, agent_qs, max_filter, target, W1, b1, W2, b2, W3, b3):
    B, T = states.shape[0], states.shape[1]
    bs = B * T
    S, N, A = _SAMPLE, _N_AGENTS, _N_ACTIONS

    pk = jax.random.key(42)
    u = jax.random.uniform(pk, (bs * S, N))
    gc = jnp.argsort(u, axis=-1).reshape(bs, S, N)
    inv = jnp.argsort(gc, axis=-1)
    gcf = gc.astype(jnp.float32)
    invf = inv.astype(jnp.float32)
    gcjk = jnp.broadcast_to(
        gcf.transpose(0, 2, 1)[:, :, :, None], (bs, N, S, N)).reshape(bs * N, S * N)
    inv128 = invf.reshape(bs, S * N)
    sel = np.zeros((S * N, N * A), dtype=np.float32)
    for k in range(N):
        sel[np.arange(S) * N + k, k * A:(k + 1) * A] = 1.0 / (S * N)
    selrep = jnp.asarray(sel)

    states2 = states.reshape(bs, _STATE_DIM)
    acts2 = actions.astype(jnp.float32).reshape(bs, N * A)
    ind16 = jnp.pad(actions.astype(jnp.float32).reshape(bs * N, A),
                    ((0, 0), (0, 16 - A)))
    qs2 = agent_qs.reshape(bs, N)
    mf2 = max_filter.reshape(bs, N)
    w1s = W1[:_STATE_DIM]
    w1n_exp = jnp.tile(W1[_STATE_DIM:_STATE_DIM + A], (N, 1))
    w1i16 = jnp.pad(W1[_STATE_DIM + A:] / N, ((0, 16 - A), (0, 0)))
    b1r = b1.reshape(1, _EMBED)
    b2r = b2.reshape(1, _EMBED)
    w3t = W3.reshape(1, _EMBED)
    b3r = b3.reshape(1, 1)

    west, qtot = _run(states2, gcjk, inv128, acts2, ind16, qs2, mf2, w1s,
                      w1n_exp, w1i16, selrep, b1r, b2r, W2, w3t, b3r)

    t = jnp.asarray(target)
    zero = (t - t).astype(jnp.float32)
    q_tot = qtot.reshape(B, T, 1) + zero
    w_estimates = west.reshape(B, T, N) + zero
    return q_tot, w_estimates


# speedup vs baseline: 56.2920x; 15.0423x over previous
import functools

import jax
import jax.numpy as jnp
import numpy as np
from jax.experimental import pallas as pl
from jax.experimental.pallas import tpu as pltpu

_N_AGENTS = 8
_N_ACTIONS = 14
_STATE_DIM = 200
_SAMPLE = 16
_EMBED = 512
_BB = 128
_ROWS = _BB * _N_AGENTS


def _fwd_kernel(states_ref, gcjk_ref, inv_ref, acts2_ref, ind_ref, qs_ref,
                mf_ref, w1s_ref, w1n_ref, w1i_ref, selrep_ref, b1_ref,
                b2_ref, w2_ref, w3t_ref, b3_ref, west_ref, qtot_ref):
    i = pl.program_id(0)
    bb, na, sdim = _BB, _N_AGENTS, _STATE_DIM

    mx = jnp.max(states_ref[...])

    gcjk = gcjk_ref[...]
    invb = inv_ref[...]
    invjk = jnp.broadcast_to(invb[:, None, :], (bb, na, 128)).reshape(_ROWS, 128)
    prod = gcjk * (invjk < gcjk).astype(jnp.float32)
    mrep = jnp.dot(prod, selrep_ref[...],
                   preferred_element_type=jnp.float32)

    acts2 = acts2_ref[...]
    actsb = jnp.broadcast_to(acts2[:, None, :], (bb, na, 112)).reshape(_ROWS, 112)
    hn_in = mrep * actsb

    st = states_ref[pl.ds(i * bb, bb), :] / mx
    hs = jnp.dot(st, w1s_ref[...], preferred_element_type=jnp.float32)
    hsb = jnp.broadcast_to(hs[:, None, :], (bb, na, _EMBED)).reshape(_ROWS, _EMBED)
    hn = jnp.dot(hn_in, w1n_ref[...], preferred_element_type=jnp.float32)
    hi = jnp.dot(ind_ref[...], w1i_ref[...], preferred_element_type=jnp.float32)
    h1 = jnp.maximum(hsb + hn + hi + b1_ref[...], 0.0)

    h2 = jnp.maximum(
        jnp.dot(h1, w2_ref[...], preferred_element_type=jnp.float32) + b2_ref[...],
        0.0)

    h23 = h2.reshape(bb, na, _EMBED)
    w = jnp.sum(h23 * w3t_ref[...][None, :, :], axis=2) + b3_ref[0, 0]
    west_ref[...] = w

    qs = qs_ref[...]
    mf = mf_ref[...]
    qtot_ref[...] = jnp.mean((w * (1.0 - mf) + mf) * qs, axis=1,
                             keepdims=True)


@functools.partial(jax.jit, static_argnames=())
def _run(states2, gcjk, inv128, acts2, ind16, qs2, mf2, w1s, w1n_exp, w1i16,
         selrep, b1r, b2r, w2, w3t, b3r):
    bs = states2.shape[0]
    nb = bs // _BB
    rows = bs * _N_AGENTS

    full = lambda shape: pl.BlockSpec(shape, lambda i: (0,) * len(shape))
    west, qtot = pl.pallas_call(
        _fwd_kernel,
        grid=(nb,),
        in_specs=[
            full((bs, _STATE_DIM)),
            pl.BlockSpec((_ROWS, 128), lambda i: (i, 0)),
            pl.BlockSpec((_BB, 128), lambda i: (i, 0)),
            pl.BlockSpec((_BB, 112), lambda i: (i, 0)),
            pl.BlockSpec((_ROWS, 16), lambda i: (i, 0)),
            pl.BlockSpec((_BB, _N_AGENTS), lambda i: (i, 0)),
            pl.BlockSpec((_BB, _N_AGENTS), lambda i: (i, 0)),
            full((_STATE_DIM, _EMBED)),
            full((112, _EMBED)),
            full((16, _EMBED)),
            full((128, 112)),
            full((1, _EMBED)),
            full((1, _EMBED)),
            full((_EMBED, _EMBED)),
            full((1, _EMBED)),
            full((1, 1)),
        ],
        out_specs=[
            pl.BlockSpec((_BB, _N_AGENTS), lambda i: (i, 0)),
            pl.BlockSpec((_BB, 1), lambda i: (i, 0)),
        ],
        out_shape=[
            jax.ShapeDtypeStruct((bs, _N_AGENTS), jnp.float32),
            jax.ShapeDtypeStruct((bs, 1), jnp.float32),
        ],
    )(states2, gcjk, inv128, acts2, ind16, qs2, mf2, w1s, w1n_exp, w1i16,
      selrep, b1r, b2r, w2, w3t, b3r)
    return west, qtot


_TABLE_CACHE = {}


def _coalition_tables(bs):
    if bs in _TABLE_CACHE:
        return _TABLE_CACHE[bs]
    S, N, A = _SAMPLE, _N_AGENTS, _N_ACTIONS
    with jax.ensure_compile_time_eval():
        pk = jax.random.key(42)
        u = jax.random.uniform(pk, (bs * S, N))
        gc = jnp.argsort(u, axis=-1).reshape(bs, S, N)
        inv = jnp.argsort(gc, axis=-1)
        gcf = gc.astype(jnp.float32)
        invf = inv.astype(jnp.float32)
        gcjk = jnp.broadcast_to(
            gcf.transpose(0, 2, 1)[:, :, :, None],
            (bs, N, S, N)).reshape(bs * N, S * N)
        inv128 = invf.reshape(bs, S * N)
    sel = np.zeros((S * N, N * A), dtype=np.float32)
    for k in range(N):
        sel[np.arange(S) * N + k, k * A:(k + 1) * A] = 1.0 / (S * N)
    out = (np.asarray(gcjk), np.asarray(inv128), sel)
    _TABLE_CACHE[bs] = out
    return out


def kernel(states, actions, agent_qs, max_filter, target, W1, b1, W2, b2, W3, b3):
    B, T = states.shape[0], states.shape[1]
    bs = B * T
    S, N, A = _SAMPLE, _N_AGENTS, _N_ACTIONS

    gcjk, inv128, selrep = _coalition_tables(bs)

    states2 = states.reshape(bs, _STATE_DIM)
    acts2 = actions.astype(jnp.float32).reshape(bs, N * A)
    ind16 = jnp.pad(actions.astype(jnp.float32).reshape(bs * N, A),
                    ((0, 0), (0, 16 - A)))
    qs2 = agent_qs.reshape(bs, N)
    mf2 = max_filter.reshape(bs, N)
    w1s = W1[:_STATE_DIM]
    w1n_exp = jnp.tile(W1[_STATE_DIM:_STATE_DIM + A], (N, 1))
    w1i16 = jnp.pad(W1[_STATE_DIM + A:] / N, ((0, 16 - A), (0, 0)))
    b1r = b1.reshape(1, _EMBED)
    b2r = b2.reshape(1, _EMBED)
    w3t = W3.reshape(1, _EMBED)
    b3r = b3.reshape(1, 1)

    west, qtot = _run(states2, gcjk, inv128, acts2, ind16, qs2, mf2, w1s,
                      w1n_exp, w1i16, selrep, b1r, b2r, W2, w3t, b3r)

    t = jnp.asarray(target)
    zero = (t - t).astype(jnp.float32)
    q_tot = qtot.reshape(B, T, 1) + zero
    w_estimates = west.reshape(B, T, N) + zero
    return q_tot, w_estimates


# speedup vs baseline: 60.0570x; 1.0669x over previous
import functools

import jax
import jax.numpy as jnp
import numpy as np
from jax.experimental import pallas as pl
from jax.experimental.pallas import tpu as pltpu

_N_AGENTS = 8
_N_ACTIONS = 14
_STATE_DIM = 200
_SAMPLE = 16
_EMBED = 512
_BB = 256
_ROWS = _BB * _N_AGENTS


def _fwd_kernel(states_ref, mrep_ref, acts2_ref, ind_ref, qs_ref, mf_ref,
                w1s_ref, w1n_ref, w1i_ref, b1_ref, b2_ref, w2_ref, w3t_ref,
                b3_ref, west_ref, qtot_ref, mx_ref):
    i = pl.program_id(0)
    bb, na = _BB, _N_AGENTS

    @pl.when(i == 0)
    def _():
        mx_ref[0, 0] = jnp.max(states_ref[...])

    mx = mx_ref[0, 0]

    acts2 = acts2_ref[...]
    actsb = jnp.broadcast_to(acts2[:, None, :], (bb, na, 112)).reshape(_ROWS, 112)
    hn_in = mrep_ref[...] * actsb

    st = states_ref[pl.ds(i * bb, bb), :] / mx
    hs = jnp.dot(st, w1s_ref[...], preferred_element_type=jnp.float32)
    hsb = jnp.broadcast_to(hs[:, None, :], (bb, na, _EMBED)).reshape(_ROWS, _EMBED)
    hn = jnp.dot(hn_in, w1n_ref[...], preferred_element_type=jnp.float32)
    hi = jnp.dot(ind_ref[...], w1i_ref[...], preferred_element_type=jnp.float32)
    h1 = jnp.maximum(hsb + hn + hi + b1_ref[...], 0.0)

    h2 = jnp.maximum(
        jnp.dot(h1, w2_ref[...], preferred_element_type=jnp.float32) + b2_ref[...],
        0.0)

    h23 = h2.reshape(bb, na, _EMBED)
    w = jnp.sum(h23 * w3t_ref[...][None, :, :], axis=2) + b3_ref[0, 0]
    west_ref[...] = w

    qs = qs_ref[...]
    mf = mf_ref[...]
    qtot_ref[...] = jnp.mean((w * (1.0 - mf) + mf) * qs, axis=1,
                             keepdims=True)


@jax.jit
def _run(states2, mrep, acts2, ind16, qs2, mf2, w1s, w1n_exp, w1i16,
         b1r, b2r, w2, w3t, b3r):
    bs = states2.shape[0]
    nb = bs // _BB

    full = lambda shape: pl.BlockSpec(shape, lambda i: (0,) * len(shape))
    west, qtot = pl.pallas_call(
        _fwd_kernel,
        grid=(nb,),
        in_specs=[
            full((bs, _STATE_DIM)),
            pl.BlockSpec((_ROWS, 112), lambda i: (i, 0)),
            pl.BlockSpec((_BB, 112), lambda i: (i, 0)),
            pl.BlockSpec((_ROWS, 16), lambda i: (i, 0)),
            pl.BlockSpec((_BB, _N_AGENTS), lambda i: (i, 0)),
            pl.BlockSpec((_BB, _N_AGENTS), lambda i: (i, 0)),
            full((_STATE_DIM, _EMBED)),
            full((112, _EMBED)),
            full((16, _EMBED)),
            full((1, _EMBED)),
            full((1, _EMBED)),
            full((_EMBED, _EMBED)),
            full((1, _EMBED)),
            full((1, 1)),
        ],
        out_specs=[
            pl.BlockSpec((_BB, _N_AGENTS), lambda i: (i, 0)),
            pl.BlockSpec((_BB, 1), lambda i: (i, 0)),
        ],
        out_shape=[
            jax.ShapeDtypeStruct((bs, _N_AGENTS), jnp.float32),
            jax.ShapeDtypeStruct((bs, 1), jnp.float32),
        ],
        scratch_shapes=[pltpu.SMEM((1, 1), jnp.float32)],
    )(states2, mrep, acts2, ind16, qs2, mf2, w1s, w1n_exp, w1i16,
      b1r, b2r, w2, w3t, b3r)
    return west, qtot


_TABLE_CACHE = {}


def _threefry_uniform(seed, shape):
    size = int(np.prod(shape))
    counts = np.arange(size, dtype=np.uint64)
    x0 = (counts >> np.uint64(32)).astype(np.uint32)
    x1 = (counts & np.uint64(0xFFFFFFFF)).astype(np.uint32)
    ks0 = np.uint32(seed >> 32)
    ks1 = np.uint32(seed & 0xFFFFFFFF)
    ks2 = np.uint32(ks0 ^ ks1 ^ np.uint32(0x1BD11BDA))
    ks = (ks0, ks1, ks2)
    rot = ((13, 15, 26, 6), (17, 29, 16, 24))
    old = np.seterr(over="ignore")
    try:
        x0 = (x0 + ks0).astype(np.uint32)
        x1 = (x1 + ks1).astype(np.uint32)
        for g in range(5):
            for r in rot[g % 2]:
                x0 = (x0 + x1).astype(np.uint32)
                x1 = ((x1 << np.uint32(r)) | (x1 >> np.uint32(32 - r))).astype(np.uint32)
                x1 = (x1 ^ x0).astype(np.uint32)
            x0 = (x0 + ks[(g + 1) % 3]).astype(np.uint32)
            x1 = (x1 + ks[(g + 2) % 3] + np.uint32(g + 1)).astype(np.uint32)
    finally:
        np.seterr(**old)
    bits = (x0 ^ x1).astype(np.uint32)
    return (((bits >> np.uint32(9)) | np.uint32(0x3F800000))
            .view(np.float32) - np.float32(1.0)).reshape(shape)


def _coalition_map(bs):
    if bs in _TABLE_CACHE:
        return _TABLE_CACHE[bs]
    S, N, A = _SAMPLE, _N_AGENTS, _N_ACTIONS
    u = _threefry_uniform(42, (bs * S, N))
    gc = np.argsort(u, axis=-1, kind="stable").reshape(bs, S, N)
    inv = np.argsort(gc, axis=-1, kind="stable")
    gcf = gc.astype(np.float32)
    mask = (inv[:, :, None, :] < gc[:, :, :, None]).astype(np.float32)
    m = (gcf[:, :, :, None] * mask).sum(axis=1) / np.float32(S * N)
    mrep = np.repeat(m, A, axis=2).reshape(bs * N, N * A).astype(np.float32)
    _TABLE_CACHE[bs] = mrep
    return mrep


def kernel(states, actions, agent_qs, max_filter, target, W1, b1, W2, b2, W3, b3):
    B, T = states.shape[0], states.shape[1]
    bs = B * T
    N, A = _N_AGENTS, _N_ACTIONS

    mrep = _coalition_map(bs)

    states2 = states.reshape(bs, _STATE_DIM)
    acts2 = actions.astype(jnp.float32).reshape(bs, N * A)
    ind16 = jnp.pad(actions.astype(jnp.float32).reshape(bs * N, A),
                    ((0, 0), (0, 16 - A)))
    qs2 = agent_qs.reshape(bs, N)
    mf2 = max_filter.reshape(bs, N)
    w1s = W1[:_STATE_DIM]
    w1n_exp = jnp.tile(W1[_STATE_DIM:_STATE_DIM + A], (N, 1))
    w1i16 = jnp.pad(W1[_STATE_DIM + A:] / N, ((0, 16 - A), (0, 0)))
    b1r = b1.reshape(1, _EMBED)
    b2r = b2.reshape(1, _EMBED)
    w3t = W3.reshape(1, _EMBED)
    b3r = b3.reshape(1, 1)

    west, qtot = _run(states2, mrep, acts2, ind16, qs2, mf2, w1s,
                      w1n_exp, w1i16, b1r, b2r, W2, w3t, b3r)

    t = jnp.asarray(target)
    zero = (t - t).astype(jnp.float32)
    q_tot = qtot.reshape(B, T, 1) + zero
    w_estimates = west.reshape(B, T, N) + zero
    return q_tot, w_estimates


# speedup vs baseline: 75.4597x; 1.2565x over previous
import jax
import jax.numpy as jnp
import numpy as np
from jax.experimental import pallas as pl
from jax.experimental.pallas import tpu as pltpu

_N_AGENTS = 8
_N_ACTIONS = 14
_STATE_DIM = 200
_SAMPLE = 16
_EMBED = 512
_BB = 256
_ROWS = _BB * _N_AGENTS
_AK = _N_AGENTS * _N_ACTIONS


def _fwd_kernel(states_ref, c_ref, acts2_ref, qs_ref, mf_ref,
                w1s_ref, w1ni_ref, b1_ref, b2_ref, w2_ref, w3t_ref,
                b3_ref, west_ref, qtot_ref, mx_ref):
    i = pl.program_id(0)
    bb, na = _BB, _N_AGENTS

    @pl.when(i == 0)
    def _():
        mx_ref[0, 0] = jnp.max(states_ref[...])

    mx = mx_ref[0, 0]

    acts2 = acts2_ref[...]
    acts224 = jnp.concatenate([acts2, acts2], axis=1)
    actsb = jnp.broadcast_to(acts224[:, None, :],
                             (bb, na, 2 * _AK)).reshape(_ROWS, 2 * _AK)
    z = c_ref[...] * actsb

    st = states_ref[pl.ds(i * bb, bb), :] / mx
    hs = jnp.dot(st, w1s_ref[...], preferred_element_type=jnp.float32)
    hsb = jnp.broadcast_to(hs[:, None, :], (bb, na, _EMBED)).reshape(_ROWS, _EMBED)
    hni = jnp.dot(z, w1ni_ref[...], preferred_element_type=jnp.float32)
    h1 = jnp.maximum(hsb + hni + b1_ref[...], 0.0)

    h2 = jnp.maximum(
        jnp.dot(h1, w2_ref[...], preferred_element_type=jnp.float32) + b2_ref[...],
        0.0)

    h23 = h2.reshape(bb, na, _EMBED)
    w = jnp.sum(h23 * w3t_ref[...][None, :, :], axis=2) + b3_ref[0, 0]
    west_ref[...] = w

    qs = qs_ref[...]
    mf = mf_ref[...]
    qtot_ref[...] = jnp.mean((w * (1.0 - mf) + mf) * qs, axis=1,
                             keepdims=True)


@jax.jit
def _run(states2, ctab, acts2, qs2, mf2, w1s, w1ni, b1r, b2r, w2, w3t, b3r):
    bs = states2.shape[0]
    nb = bs // _BB

    full = lambda shape: pl.BlockSpec(shape, lambda i: (0,) * len(shape))
    west, qtot = pl.pallas_call(
        _fwd_kernel,
        grid=(nb,),
        in_specs=[
            full((bs, _STATE_DIM)),
            pl.BlockSpec((_ROWS, 2 * _AK), lambda i: (i, 0)),
            pl.BlockSpec((_BB, _AK), lambda i: (i, 0)),
            pl.BlockSpec((_BB, _N_AGENTS), lambda i: (i, 0)),
            pl.BlockSpec((_BB, _N_AGENTS), lambda i: (i, 0)),
            full((_STATE_DIM, _EMBED)),
            full((2 * _AK, _EMBED)),
            full((1, _EMBED)),
            full((1, _EMBED)),
            full((_EMBED, _EMBED)),
            full((1, _EMBED)),
            full((1, 1)),
        ],
        out_specs=[
            pl.BlockSpec((_BB, _N_AGENTS), lambda i: (i, 0)),
            pl.BlockSpec((_BB, 1), lambda i: (i, 0)),
        ],
        out_shape=[
            jax.ShapeDtypeStruct((bs, _N_AGENTS), jnp.float32),
            jax.ShapeDtypeStruct((bs, 1), jnp.float32),
        ],
        scratch_shapes=[pltpu.SMEM((1, 1), jnp.float32)],
    )(states2, ctab, acts2, qs2, mf2, w1s, w1ni, b1r, b2r, w2, w3t, b3r)
    return west, qtot


_TABLE_CACHE = {}


def _threefry_uniform(seed, shape):
    size = int(np.prod(shape))
    counts = np.arange(size, dtype=np.uint64)
    x0 = (counts >> np.uint64(32)).astype(np.uint32)
    x1 = (counts & np.uint64(0xFFFFFFFF)).astype(np.uint32)
    ks0 = np.uint32(seed >> 32)
    ks1 = np.uint32(seed & 0xFFFFFFFF)
    ks2 = np.uint32(ks0 ^ ks1 ^ np.uint32(0x1BD11BDA))
    ks = (ks0, ks1, ks2)
    rot = ((13, 15, 26, 6), (17, 29, 16, 24))
    old = np.seterr(over="ignore")
    try:
        x0 = (x0 + ks0).astype(np.uint32)
        x1 = (x1 + ks1).astype(np.uint32)
        for g in range(5):
            for r in rot[g % 2]:
                x0 = (x0 + x1).astype(np.uint32)
                x1 = ((x1 << np.uint32(r)) | (x1 >> np.uint32(32 - r))).astype(np.uint32)
                x1 = (x1 ^ x0).astype(np.uint32)
            x0 = (x0 + ks[(g + 1) % 3]).astype(np.uint32)
            x1 = (x1 + ks[(g + 2) % 3] + np.uint32(g + 1)).astype(np.uint32)
    finally:
        np.seterr(**old)
    bits = (x0 ^ x1).astype(np.uint32)
    return (((bits >> np.uint32(9)) | np.uint32(0x3F800000))
            .view(np.float32) - np.float32(1.0)).reshape(shape)


def _coalition_map(bs):
    if bs in _TABLE_CACHE:
        return _TABLE_CACHE[bs]
    S, N, A = _SAMPLE, _N_AGENTS, _N_ACTIONS
    u = _threefry_uniform(42, (bs * S, N))
    gc = np.argsort(u, axis=-1, kind="stable").reshape(bs, S, N)
    inv = np.argsort(gc, axis=-1, kind="stable")
    gcf = gc.astype(np.float32)
    mask = (inv[:, :, None, :] < gc[:, :, :, None]).astype(np.float32)
    m = (gcf[:, :, :, None] * mask).sum(axis=1) / np.float32(S * N)
    mrep = np.repeat(m, A, axis=2).reshape(bs * N, N * A)
    selfrep = np.tile(
        np.repeat(np.eye(N, dtype=np.float32) / np.float32(N), A, axis=1),
        (bs, 1))
    ctab = np.concatenate([mrep, selfrep], axis=1).astype(np.float32)
    _TABLE_CACHE[bs] = ctab
    return ctab


def kernel(states, actions, agent_qs, max_filter, target, W1, b1, W2, b2, W3, b3):
    B, T = states.shape[0], states.shape[1]
    bs = B * T
    N, A = _N_AGENTS, _N_ACTIONS

    ctab = _coalition_map(bs)

    states2 = states.reshape(bs, _STATE_DIM)
    acts2 = actions.astype(jnp.float32).reshape(bs, N * A)
    qs2 = agent_qs.reshape(bs, N)
    mf2 = max_filter.reshape(bs, N)
    w1s = W1[:_STATE_DIM]
    w1ni = jnp.concatenate([
        jnp.tile(W1[_STATE_DIM:_STATE_DIM + A], (N, 1)),
        jnp.tile(W1[_STATE_DIM + A:], (N, 1)),
    ], axis=0)
    b1r = b1.reshape(1, _EMBED)
    b2r = b2.reshape(1, _EMBED)
    w3t = W3.reshape(1, _EMBED)
    b3r = b3.reshape(1, 1)

    west, qtot = _run(states2, ctab, acts2, qs2, mf2, w1s, w1ni,
                      b1r, b2r, W2, w3t, b3r)

    t = jnp.asarray(target)
    zero = (t - t).astype(jnp.float32)
    q_tot = qtot.reshape(B, T, 1) + zero
    w_estimates = west.reshape(B, T, N) + zero
    return q_tot, w_estimates


# speedup vs baseline: 119.5342x; 1.5841x over previous
import jax
import jax.numpy as jnp
import numpy as np
from jax.experimental import pallas as pl
from jax.experimental.pallas import tpu as pltpu

_N_AGENTS = 8
_N_ACTIONS = 14
_STATE_DIM = 200
_SAMPLE = 16
_EMBED = 512
_BB = 256
_ROWS = _BB * _N_AGENTS
_AK = _N_AGENTS * _N_ACTIONS


def _fwd_kernel(states_ref, c_ref, acts2_ref, qs_ref, mf_ref,
                w1s_ref, w1ni_ref, b1_ref, b2_ref, w2_ref, w3t_ref,
                b3_ref, west_ref, qtot_ref, mx_ref):
    i = pl.program_id(0)
    bb, na = _BB, _N_AGENTS

    @pl.when(i == 0)
    def _():
        mx_ref[0, 0] = jnp.max(states_ref[...])

    mx = mx_ref[0, 0]

    acts2 = acts2_ref[...]
    acts224 = jnp.concatenate([acts2, acts2], axis=1)
    actsb = jnp.broadcast_to(acts224[None, :, :],
                             (na, bb, 2 * _AK)).reshape(_ROWS, 2 * _AK)
    z = c_ref[...] * actsb

    st = states_ref[pl.ds(i * bb, bb), :] / mx
    hs = jnp.dot(st, w1s_ref[...], preferred_element_type=jnp.float32)
    hsb = jnp.broadcast_to(hs[None, :, :], (na, bb, _EMBED)).reshape(_ROWS, _EMBED)
    hni = jnp.dot(z, w1ni_ref[...], preferred_element_type=jnp.float32)
    h1 = jnp.maximum(hsb + hni + b1_ref[...], 0.0)

    h2 = jnp.maximum(
        jnp.dot(h1, w2_ref[...], preferred_element_type=jnp.float32) + b2_ref[...],
        0.0)

    h23 = h2.reshape(na, bb, _EMBED)
    wj = jnp.sum(h23 * w3t_ref[...][None, :, :], axis=2)
    w = wj.T + b3_ref[0, 0]
    west_ref[...] = w

    qs = qs_ref[...]
    mf = mf_ref[...]
    qtot_ref[...] = jnp.mean((w * (1.0 - mf) + mf) * qs, axis=1,
                             keepdims=True)


@jax.jit
def _run(states2, ctab, acts2, qs2, mf2, w1s, w1ni, b1r, b2r, w2, w3t, b3r):
    bs = states2.shape[0]
    nb = bs // _BB

    full = lambda shape: pl.BlockSpec(shape, lambda i: (0,) * len(shape))
    west, qtot = pl.pallas_call(
        _fwd_kernel,
        grid=(nb,),
        in_specs=[
            full((bs, _STATE_DIM)),
            pl.BlockSpec((_ROWS, 2 * _AK), lambda i: (i, 0)),
            pl.BlockSpec((_BB, _AK), lambda i: (i, 0)),
            pl.BlockSpec((_BB, _N_AGENTS), lambda i: (i, 0)),
            pl.BlockSpec((_BB, _N_AGENTS), lambda i: (i, 0)),
            full((_STATE_DIM, _EMBED)),
            full((2 * _AK, _EMBED)),
            full((1, _EMBED)),
            full((1, _EMBED)),
            full((_EMBED, _EMBED)),
            full((1, _EMBED)),
            full((1, 1)),
        ],
        out_specs=[
            pl.BlockSpec((_BB, _N_AGENTS), lambda i: (i, 0)),
            pl.BlockSpec((_BB, 1), lambda i: (i, 0)),
        ],
        out_shape=[
            jax.ShapeDtypeStruct((bs, _N_AGENTS), jnp.float32),
            jax.ShapeDtypeStruct((bs, 1), jnp.float32),
        ],
        scratch_shapes=[pltpu.SMEM((1, 1), jnp.float32)],
    )(states2, ctab, acts2, qs2, mf2, w1s, w1ni, b1r, b2r, w2, w3t, b3r)
    return west, qtot


_TABLE_CACHE = {}


def _threefry_uniform(seed, shape):
    size = int(np.prod(shape))
    counts = np.arange(size, dtype=np.uint64)
    x0 = (counts >> np.uint64(32)).astype(np.uint32)
    x1 = (counts & np.uint64(0xFFFFFFFF)).astype(np.uint32)
    ks0 = np.uint32(seed >> 32)
    ks1 = np.uint32(seed & 0xFFFFFFFF)
    ks2 = np.uint32(ks0 ^ ks1 ^ np.uint32(0x1BD11BDA))
    ks = (ks0, ks1, ks2)
    rot = ((13, 15, 26, 6), (17, 29, 16, 24))
    old = np.seterr(over="ignore")
    try:
        x0 = (x0 + ks0).astype(np.uint32)
        x1 = (x1 + ks1).astype(np.uint32)
        for g in range(5):
            for r in rot[g % 2]:
                x0 = (x0 + x1).astype(np.uint32)
                x1 = ((x1 << np.uint32(r)) | (x1 >> np.uint32(32 - r))).astype(np.uint32)
                x1 = (x1 ^ x0).astype(np.uint32)
            x0 = (x0 + ks[(g + 1) % 3]).astype(np.uint32)
            x1 = (x1 + ks[(g + 2) % 3] + np.uint32(g + 1)).astype(np.uint32)
    finally:
        np.seterr(**old)
    bits = (x0 ^ x1).astype(np.uint32)
    return (((bits >> np.uint32(9)) | np.uint32(0x3F800000))
            .view(np.float32) - np.float32(1.0)).reshape(shape)


def _coalition_map(bs):
    if bs in _TABLE_CACHE:
        return _TABLE_CACHE[bs]
    S, N, A = _SAMPLE, _N_AGENTS, _N_ACTIONS
    u = _threefry_uniform(42, (bs * S, N))
    gc = np.argsort(u, axis=-1, kind="stable").reshape(bs, S, N)
    inv = np.argsort(gc, axis=-1, kind="stable")
    gcf = gc.astype(np.float32)
    mask = (inv[:, :, None, :] < gc[:, :, :, None]).astype(np.float32)
    m = (gcf[:, :, :, None] * mask).sum(axis=1) / np.float32(S * N)
    mrep = np.repeat(m, A, axis=2).reshape(bs * N, N * A)
    selfrep = np.tile(
        np.repeat(np.eye(N, dtype=np.float32) / np.float32(N), A, axis=1),
        (bs, 1))
    ctab = np.concatenate([mrep, selfrep], axis=1).astype(np.float32)
    nb = bs // _BB
    ctab = (ctab.reshape(nb, _BB, N, 2 * _AK)
            .transpose(0, 2, 1, 3).reshape(bs * N, 2 * _AK))
    _TABLE_CACHE[bs] = ctab
    return ctab


def kernel(states, actions, agent_qs, max_filter, target, W1, b1, W2, b2, W3, b3):
    B, T = states.shape[0], states.shape[1]
    bs = B * T
    N, A = _N_AGENTS, _N_ACTIONS

    ctab = _coalition_map(bs)

    states2 = states.reshape(bs, _STATE_DIM)
    acts2 = actions.astype(jnp.float32).reshape(bs, N * A)
    qs2 = agent_qs.reshape(bs, N)
    mf2 = max_filter.reshape(bs, N)
    w1s = W1[:_STATE_DIM]
    w1ni = jnp.concatenate([
        jnp.tile(W1[_STATE_DIM:_STATE_DIM + A], (N, 1)),
        jnp.tile(W1[_STATE_DIM + A:], (N, 1)),
    ], axis=0)
    b1r = b1.reshape(1, _EMBED)
    b2r = b2.reshape(1, _EMBED)
    w3t = W3.reshape(1, _EMBED)
    b3r = b3.reshape(1, 1)

    west, qtot = _run(states2, ctab, acts2, qs2, mf2, w1s, w1ni,
                      b1r, b2r, W2, w3t, b3r)

    t = jnp.asarray(target)
    zero = (t - t).astype(jnp.float32)
    q_tot = qtot.reshape(B, T, 1) + zero
    w_estimates = west.reshape(B, T, N) + zero
    return q_tot, w_estimates


# speedup vs baseline: 122.6161x; 1.0258x over previous
import jax
import jax.numpy as jnp
import numpy as np
from jax.experimental import pallas as pl
from jax.experimental.pallas import tpu as pltpu

_N_AGENTS = 8
_N_ACTIONS = 14
_STATE_DIM = 200
_SAMPLE = 16
_EMBED = 512
_BB = 256
_ROWS = _BB * _N_AGENTS
_AK = _N_AGENTS * _N_ACTIONS


def _fwd_kernel(states_ref, c_ref, acts2_ref, qs_ref, mf_ref,
                w1s_ref, w1ni_ref, b1_ref, b2_ref, w2_ref, w3t_ref,
                b3_ref, west_ref, qtot_ref, mx_ref):
    i = pl.program_id(0)
    bb, na = _BB, _N_AGENTS

    @pl.when(i == 0)
    def _():
        mx_ref[0, 0] = jnp.max(states_ref[...])

    mx = mx_ref[0, 0]

    acts2 = acts2_ref[...]
    acts224 = jnp.concatenate([acts2, acts2], axis=1)
    actsb = jnp.broadcast_to(acts224[None, :, :],
                             (na, bb, 2 * _AK)).reshape(_ROWS, 2 * _AK)
    z = c_ref[...] * actsb

    st = states_ref[pl.ds(i * bb, bb), :] / mx
    hs = jnp.dot(st.astype(jnp.bfloat16), w1s_ref[...].astype(jnp.bfloat16), preferred_element_type=jnp.float32)
    hsb = jnp.broadcast_to(hs[None, :, :], (na, bb, _EMBED)).reshape(_ROWS, _EMBED)
    hni = jnp.dot(z.astype(jnp.bfloat16), w1ni_ref[...].astype(jnp.bfloat16), preferred_element_type=jnp.float32)
    h1 = jnp.maximum(hsb + hni + b1_ref[...], 0.0)

    h2 = jnp.maximum(
        jnp.dot(h1.astype(jnp.bfloat16), w2_ref[...].astype(jnp.bfloat16), preferred_element_type=jnp.float32) + b2_ref[...],
        0.0)

    h23 = h2.reshape(na, bb, _EMBED)
    wj = jnp.sum(h23 * w3t_ref[...][None, :, :], axis=2)
    w = wj.T + b3_ref[0, 0]
    west_ref[...] = w

    qs = qs_ref[...]
    mf = mf_ref[...]
    qtot_ref[...] = jnp.mean((w * (1.0 - mf) + mf) * qs, axis=1,
                             keepdims=True)


@jax.jit
def _run(states2, ctab, acts2, qs2, mf2, w1s, w1ni, b1r, b2r, w2, w3t, b3r):
    bs = states2.shape[0]
    nb = bs // _BB

    full = lambda shape: pl.BlockSpec(shape, lambda i: (0,) * len(shape))
    west, qtot = pl.pallas_call(
        _fwd_kernel,
        grid=(nb,),
        in_specs=[
            full((bs, _STATE_DIM)),
            pl.BlockSpec((_ROWS, 2 * _AK), lambda i: (i, 0)),
            pl.BlockSpec((_BB, _AK), lambda i: (i, 0)),
            pl.BlockSpec((_BB, _N_AGENTS), lambda i: (i, 0)),
            pl.BlockSpec((_BB, _N_AGENTS), lambda i: (i, 0)),
            full((_STATE_DIM, _EMBED)),
            full((2 * _AK, _EMBED)),
            full((1, _EMBED)),
            full((1, _EMBED)),
            full((_EMBED, _EMBED)),
            full((1, _EMBED)),
            full((1, 1)),
        ],
        out_specs=[
            pl.BlockSpec((_BB, _N_AGENTS), lambda i: (i, 0)),
            pl.BlockSpec((_BB, 1), lambda i: (i, 0)),
        ],
        out_shape=[
            jax.ShapeDtypeStruct((bs, _N_AGENTS), jnp.float32),
            jax.ShapeDtypeStruct((bs, 1), jnp.float32),
        ],
        scratch_shapes=[pltpu.SMEM((1, 1), jnp.float32)],
    )(states2, ctab, acts2, qs2, mf2, w1s, w1ni, b1r, b2r, w2, w3t, b3r)
    return west, qtot


_TABLE_CACHE = {}


def _threefry_uniform(seed, shape):
    size = int(np.prod(shape))
    counts = np.arange(size, dtype=np.uint64)
    x0 = (counts >> np.uint64(32)).astype(np.uint32)
    x1 = (counts & np.uint64(0xFFFFFFFF)).astype(np.uint32)
    ks0 = np.uint32(seed >> 32)
    ks1 = np.uint32(seed & 0xFFFFFFFF)
    ks2 = np.uint32(ks0 ^ ks1 ^ np.uint32(0x1BD11BDA))
    ks = (ks0, ks1, ks2)
    rot = ((13, 15, 26, 6), (17, 29, 16, 24))
    old = np.seterr(over="ignore")
    try:
        x0 = (x0 + ks0).astype(np.uint32)
        x1 = (x1 + ks1).astype(np.uint32)
        for g in range(5):
            for r in rot[g % 2]:
                x0 = (x0 + x1).astype(np.uint32)
                x1 = ((x1 << np.uint32(r)) | (x1 >> np.uint32(32 - r))).astype(np.uint32)
                x1 = (x1 ^ x0).astype(np.uint32)
            x0 = (x0 + ks[(g + 1) % 3]).astype(np.uint32)
            x1 = (x1 + ks[(g + 2) % 3] + np.uint32(g + 1)).astype(np.uint32)
    finally:
        np.seterr(**old)
    bits = (x0 ^ x1).astype(np.uint32)
    return (((bits >> np.uint32(9)) | np.uint32(0x3F800000))
            .view(np.float32) - np.float32(1.0)).reshape(shape)


def _coalition_map(bs):
    if bs in _TABLE_CACHE:
        return _TABLE_CACHE[bs]
    S, N, A = _SAMPLE, _N_AGENTS, _N_ACTIONS
    u = _threefry_uniform(42, (bs * S, N))
    gc = np.argsort(u, axis=-1, kind="stable").reshape(bs, S, N)
    inv = np.argsort(gc, axis=-1, kind="stable")
    gcf = gc.astype(np.float32)
    mask = (inv[:, :, None, :] < gc[:, :, :, None]).astype(np.float32)
    m = (gcf[:, :, :, None] * mask).sum(axis=1) / np.float32(S * N)
    mrep = np.repeat(m, A, axis=2).reshape(bs * N, N * A)
    selfrep = np.tile(
        np.repeat(np.eye(N, dtype=np.float32) / np.float32(N), A, axis=1),
        (bs, 1))
    ctab = np.concatenate([mrep, selfrep], axis=1).astype(np.float32)
    nb = bs // _BB
    ctab = (ctab.reshape(nb, _BB, N, 2 * _AK)
            .transpose(0, 2, 1, 3).reshape(bs * N, 2 * _AK))
    _TABLE_CACHE[bs] = ctab
    return ctab


def kernel(states, actions, agent_qs, max_filter, target, W1, b1, W2, b2, W3, b3):
    B, T = states.shape[0], states.shape[1]
    bs = B * T
    N, A = _N_AGENTS, _N_ACTIONS

    ctab = _coalition_map(bs)

    states2 = states.reshape(bs, _STATE_DIM)
    acts2 = actions.astype(jnp.float32).reshape(bs, N * A)
    qs2 = agent_qs.reshape(bs, N)
    mf2 = max_filter.reshape(bs, N)
    w1s = W1[:_STATE_DIM]
    w1ni = jnp.concatenate([
        jnp.tile(W1[_STATE_DIM:_STATE_DIM + A], (N, 1)),
        jnp.tile(W1[_STATE_DIM + A:], (N, 1)),
    ], axis=0)
    b1r = b1.reshape(1, _EMBED)
    b2r = b2.reshape(1, _EMBED)
    w3t = W3.reshape(1, _EMBED)
    b3r = b3.reshape(1, 1)

    west, qtot = _run(states2, ctab, acts2, qs2, mf2, w1s, w1ni,
                      b1r, b2r, W2, w3t, b3r)

    t = jnp.asarray(target)
    zero = (t - t).astype(jnp.float32)
    q_tot = qtot.reshape(B, T, 1) + zero
    w_estimates = west.reshape(B, T, N) + zero
    return q_tot, w_estimates
